# baseline (speedup 1.0000x reference)
import functools

import jax
import jax.numpy as jnp
from jax import lax
from jax.experimental import pallas as pl
from jax.experimental.pallas import tpu as pltpu
from jax.experimental.pallas import tpu_sc as plsc

_N, _E, _D, _L, _G = 10000, 320000, 128, 4, 16
_NC, _NS = 2, 16
_NW = _NC * _NS
_C = 128
_ROWS = 2560
_EPAD = _ROWS * _C
_RPT = _ROWS // _NW
_NB = 4
_NGRP = _RPT // _NB
_NPAD0 = 10112
_BN = 632
_BNE = 640


def _sc_mesh():
  return plsc.VectorSubcoreMesh(core_axis_name="c", subcore_axis_name="s",
                                num_cores=_NC, num_subcores=_NS)


_SC_PARAMS = pltpu.CompilerParams(needs_layout_passes=False)


def _shift_body(s_ref, d_ref, os_ref, od_ref):
  for t in range(_L):
    os_ref[t] = lax.shift_right_logical(s_ref[...], t)
    od_ref[t] = lax.shift_right_logical(d_ref[...], t)


def _shift_call(srcp, dstp):
  grid = _ROWS // _BNE
  return pl.pallas_call(
      _shift_body,
      grid=(grid,),
      in_specs=[pl.BlockSpec((_BNE, _C), lambda i: (i, 0)),
                pl.BlockSpec((_BNE, _C), lambda i: (i, 0))],
      out_specs=[pl.BlockSpec((_L, _BNE, _C), lambda i: (0, i, 0)),
                 pl.BlockSpec((_L, _BNE, _C), lambda i: (0, i, 0))],
      out_shape=[jax.ShapeDtypeStruct((_L, _ROWS, _C), jnp.int32),
                 jax.ShapeDtypeStruct((_L, _ROWS, _C), jnp.int32)],
  )(srcp, dstp)


def _acc_sched(npad):
  chunk = 8 * (-(-npad // (8 * _NS)))
  full = npad // chunk
  rem = npad - full * chunk
  return chunk, full, rem


def _acc_zero(zer_hbm, acc, sid, chunk, full, rem):
  @pl.when(sid < full)
  def _zero():
    pltpu.sync_copy(zer_hbm, acc.at[pl.ds(sid * chunk, chunk)])

  if rem:
    @pl.when(sid == full)
    def _zero_rem():
      pltpu.sync_copy(zer_hbm.at[pl.ds(0, rem)],
                      acc.at[pl.ds(full * chunk, rem)])


def _acc_out(acc, out_hbm, cid, sid, chunk, full, rem):
  @pl.when(sid < full)
  def _out():
    pltpu.sync_copy(acc.at[pl.ds(sid * chunk, chunk)],
                    out_hbm.at[cid, pl.ds(sid * chunk, chunk)])

  if rem:
    @pl.when(sid == full)
    def _out_rem():
      pltpu.sync_copy(acc.at[pl.ds(full * chunk, rem)],
                      out_hbm.at[cid, pl.ds(full * chunk, rem)])


def _stage_x(x_hbm, xsh, sid, chunk, full, rem):
  @pl.when(sid < full)
  def _st():
    pltpu.sync_copy(x_hbm.at[pl.ds(sid * chunk, chunk)],
                    xsh.at[pl.ds(sid * chunk, chunk)])

  if rem:
    @pl.when(sid == full)
    def _st_rem():
      pltpu.sync_copy(x_hbm.at[pl.ds(full * chunk, rem)],
                      xsh.at[pl.ds(full * chunk, rem)])


def _sc_agg_call(npad, xp, src_arr, dst_r, zer2, *, lean, stage,
                 split=None, stream_dst=False, single=False):
  chunk, full, rem = _acc_sched(npad)
  if lean:
    nb = 3 if stream_dst else 2
  else:
    nb = _NB
  rpt = _ROWS // _NS if single else _RPT
  rpt_max = split[0] if split else rpt
  nc_out = 1 if single else _NC

  scratch = []
  if lean:
    if not stream_dst:
      scratch += [pltpu.VMEM((rpt_max, _C), jnp.int32)]
    scratch += [pltpu.VMEM((_C,), jnp.int32)] * nb
    if stream_dst:
      scratch += [pltpu.VMEM((_C,), jnp.int32)] * nb
  else:
    scratch += [pltpu.VMEM((rpt, _C), jnp.int32),
                pltpu.VMEM((rpt, _C), jnp.int32)]
  scratch += [pltpu.VMEM((_C, _D), jnp.float32)] * nb
  scratch += [pltpu.VMEM_SHARED((npad, _D), jnp.float32)]
  if stage:
    scratch += [pltpu.VMEM_SHARED((npad, _D), jnp.float32)]
  scratch += [pltpu.SemaphoreType.DMA] * ((3 if lean else 2) * nb)

  @functools.partial(
      pl.kernel,
      out_type=jax.ShapeDtypeStruct((nc_out, npad, _D), jnp.float32),
      mesh=_sc_mesh(),
      compiler_params=_SC_PARAMS,
      scratch_types=scratch)
  def k(x_hbm, src_hbm, dst_hbm, zer_hbm, out_hbm, *rest):
    p = 0
    didx = di = sidx = None
    if lean:
      if not stream_dst:
        didx = rest[p]; p += 1
      si = rest[p:p + nb]; p += nb
      if stream_dst:
        di = rest[p:p + nb]; p += nb
    else:
      sidx, didx = rest[p], rest[p + 1]; p += 2
    rows = rest[p:p + nb]; p += nb
    acc = rest[p]; p += 1
    xsh = rest[p] if stage else None
    if stage:
      p += 1
    sems = rest[p:]
    if lean:
      isem, gsem, ssem = sems[:nb], sems[nb:2 * nb], sems[2 * nb:]
    else:
      gsem, ssem = sems[:nb], sems[nb:]

    cid = lax.axis_index("c")
    sid = lax.axis_index("s")
    wid = sid * _NC + cid

    if split:
      rpt_f, rpt_s, fcid = split
      is_fast = cid == fcid
      rpt_my = jnp.where(is_fast, rpt_f, rpt_s)
      base_e = jnp.where(is_fast, sid * rpt_f, _NS * rpt_f + sid * rpt_s)
      ngrp_my = (rpt_my + nb - 1) // nb
      ragged = True
    elif single:
      rpt_my = rpt
      base_e = sid * rpt
      ngrp_my = (rpt + nb - 1) // nb
      ragged = rpt % nb != 0
    else:
      rpt_my = rpt
      base_e = wid * rpt
      ngrp_my = (rpt + nb - 1) // nb
      ragged = rpt % nb != 0

    def dref(g, b):
      return di[b] if stream_dst else didx.at[g]

    def scat_wait(b):
      d = di[b] if stream_dst else didx.at[0]
      pltpu.make_async_copy(rows[b], acc.at[d], ssem[b]).wait()

    def work():
      with jax.named_scope("agg_prep"):
        if didx is not None:
          pltpu.sync_copy(dst_hbm.at[pl.ds(base_e, rpt)], didx)
        if not lean:
          pltpu.sync_copy(src_hbm.at[pl.ds(base_e, rpt)], sidx)
        _acc_zero(zer_hbm, acc, sid, chunk, full, rem)
        if stage:
          _stage_x(x_hbm, xsh, sid, chunk, full, rem)
        plsc.subcore_barrier()
      gsrc = xsh if stage else x_hbm

      if lean:
        def idx_issue(g, b):
          pltpu.async_copy(src_hbm.at[pl.ds((base_e + g) * _C, _C)], si[b],
                           isem[b])
          if stream_dst:
            pltpu.async_copy(dst_hbm.at[pl.ds((base_e + g) * _C, _C)],
                             di[b], isem[b])

        def idx_wait(b):
          pltpu.make_async_copy(src_hbm.at[pl.ds(0, _C)], si[b],
                                isem[b]).wait()
          if stream_dst:
            pltpu.make_async_copy(src_hbm.at[pl.ds(0, _C)], di[b],
                                  isem[b]).wait()

        for b in range(nb):
          idx_issue(b, b)
        for b in range(nb):
          idx_wait(b)
          pltpu.async_copy(gsrc.at[si[b]], rows[b], gsem[b]).wait()
          idx_issue(b + nb, b)
          pltpu.async_copy(rows[b], acc.at[dref(b, b)], ssem[b], add=True)

        def group(k_i, carry):
          for b in range(nb):
            g = nb * k_i + b

            def slot_body():
              scat_wait(b)
              idx_wait(b)
              pltpu.async_copy(gsrc.at[si[b]], rows[b], gsem[b]).wait()
              g_next = g + nb

              @pl.when(g_next < rpt_my)
              def _prefetch_idx():
                idx_issue(g_next, b)

              pltpu.async_copy(rows[b], acc.at[dref(g, b)], ssem[b],
                               add=True)

            if ragged:
              pl.when(g < rpt_my)(slot_body)
            else:
              slot_body()
          return carry

        lax.fori_loop(1, ngrp_my, group, 0)
        for b in range(nb):
          scat_wait(b)
      else:
        for b in range(nb):
          pltpu.async_copy(gsrc.at[sidx.at[b]], rows[b], gsem[b])

        def group(k_i, carry):
          scat = []
          for b in range(nb):
            g = nb * k_i + b
            pltpu.make_async_copy(gsrc.at[sidx.at[g]], rows[b],
                                  gsem[b]).wait()
            scat.append(
                pltpu.async_copy(rows[b], acc.at[didx.at[g]], ssem[b],
                                 add=True))
          for b in range(nb):
            g_next = nb * k_i + b + nb
            scat[b].wait()

            @pl.when(g_next < rpt)
            def _prefetch():
              pltpu.async_copy(gsrc.at[sidx.at[g_next]], rows[b], gsem[b])

          return carry

        lax.fori_loop(0, ngrp_my, group, 0)

      with jax.named_scope("agg_out"):
        plsc.subcore_barrier()
        _acc_out(acc, out_hbm, 0 if single else cid, sid, chunk, full, rem)

    if single:
      pl.when(cid == 0)(work)
    else:
      work()

  return k(xp, src_arr, dst_r, zer2)


def _dense1_call(xp, aggp, wt, bt, wsc, bsr, nreal):
  npad = xp.shape[0]
  grid = npad // _BN

  def body(x_ref, a_ref, w_ref, b_ref, ws_ref, bs_ref, h_ref, s_ref):
    i = pl.program_id(0)
    xa = x_ref[...]
    for q in range(a_ref.shape[0]):
      xa = xa + a_ref[q]
    h = jnp.dot(xa, w_ref[...], preferred_element_type=jnp.float32)
    h = jnp.maximum(h + b_ref[...], 0.0)
    rows = i * _BN + lax.broadcasted_iota(jnp.int32, (_BN, 1), 0)
    h = jnp.where(rows < nreal, h, 0.0)
    h_ref[...] = h
    s_ref[...] = jnp.dot(h, ws_ref[...],
                         preferred_element_type=jnp.float32) + bs_ref[...]

  return pl.pallas_call(
      body,
      grid=(grid,),
      in_specs=[pl.BlockSpec((_BN, _D), lambda i: (i, 0)),
                pl.BlockSpec((aggp.shape[0], _BN, _D), lambda i: (0, i, 0)),
                pl.BlockSpec((_D, _D), lambda i: (0, 0)),
                pl.BlockSpec((1, _D), lambda i: (0, 0)),
                pl.BlockSpec((_D, _D), lambda i: (0, 0)),
                pl.BlockSpec((1, _D), lambda i: (0, 0))],
      out_specs=[pl.BlockSpec((_BN, _D), lambda i: (i, 0)),
                 pl.BlockSpec((_BN, _D), lambda i: (i, 0))],
      out_shape=[jax.ShapeDtypeStruct((npad, _D), jnp.float32),
                 jax.ShapeDtypeStruct((npad, _D), jnp.float32)],
  )(xp, aggp, wt, bt, wsc, bsr)


def _sc_score(npad, s1, s2, src_l, dst_l, zer1):
  npadc = 128 * (-(-npad // 128))

  @functools.partial(
      pl.kernel,
      out_type=jax.ShapeDtypeStruct((_NC * npadc,), jnp.float32),
      mesh=_sc_mesh(),
      compiler_params=_SC_PARAMS,
      scratch_types=[
          pltpu.VMEM((npad,), jnp.float32),
          pltpu.VMEM((npad,), jnp.float32),
          pltpu.VMEM((_RPT, _C), jnp.int32),
          pltpu.VMEM((_RPT, _C), jnp.int32),
          pltpu.VMEM((_RPT, _C), jnp.float32),
          pltpu.VMEM_SHARED((npadc,), jnp.float32),
          pltpu.SemaphoreType.DMA,
      ])
  def k(s1_hbm, s2_hbm, src_hbm, dst_hbm, zer_hbm, out_hbm,
        s1v, s2v, sidx, didx, sig, acc, ssem):
    cid = lax.axis_index("c")
    sid = lax.axis_index("s")
    wid = sid * _NC + cid
    pltpu.sync_copy(s1_hbm, s1v)
    pltpu.sync_copy(s2_hbm, s2v)
    pltpu.sync_copy(src_hbm.at[pl.ds(wid * _RPT, _RPT)], sidx)
    pltpu.sync_copy(dst_hbm.at[pl.ds(wid * _RPT, _RPT)], didx)

    @pl.when(sid == 0)
    def _zero():
      pltpu.sync_copy(zer_hbm, acc)

    plsc.subcore_barrier()

    def body(i, carry):
      for j in range(_C // 16):
        s_idx = sidx[i, pl.ds(16 * j, 16)]
        d_idx = didx[i, pl.ds(16 * j, 16)]
        v1 = plsc.load_gather(s1v, [s_idx])
        v2 = plsc.load_gather(s2v, [d_idx])
        z = v1 + v2
        sig[i, pl.ds(16 * j, 16)] = 1.0 / (1.0 + jnp.exp(-z))
      pltpu.async_copy(sig.at[i], acc.at[didx.at[i]], ssem, add=True)
      return carry

    lax.fori_loop(0, _RPT, body, 0)

    def drain(i, carry):
      pltpu.make_async_copy(sig.at[0], acc.at[didx.at[0]], ssem).wait()
      return carry

    lax.fori_loop(0, _RPT, drain, 0)
    plsc.subcore_barrier()

    @pl.when(sid == 0)
    def _out():
      pltpu.sync_copy(acc, out_hbm.at[pl.ds(cid * npadc, npadc)])

  return k(s1, s2, src_l, dst_l, zer1)


def _combine_call(hr, nwp):
  n2 = hr.shape[0]
  grid = n2 // _BN

  def body(h_ref, nw_ref, o_ref):
    nw = nw_ref[0] + nw_ref[1]
    w0 = 1.0 + nw[:, 0:1]
    w1 = 1.0 + nw[:, 1:2]
    o_ref[...] = h_ref[:, :_D] * w0 + h_ref[:, _D:] * w1

  return pl.pallas_call(
      body,
      grid=(grid,),
      in_specs=[pl.BlockSpec((_BN, 2 * _D), lambda i: (i, 0)),
                pl.BlockSpec((_NC, _BN, 2), lambda i: (0, i, 0))],
      out_specs=pl.BlockSpec((_BN, _D), lambda i: (i, 0)),
      out_shape=jax.ShapeDtypeStruct((n2, _D), jnp.float32),
  )(hr, nwp)


def _pool_call(xp, b4p):
  n4 = xp.shape[0]

  def body(x_ref, b_ref, o_ref):
    iota = lax.broadcasted_iota(jnp.int32, (_G, n4), 0)
    oh = (b_ref[...] == iota).astype(jnp.float32)
    o_ref[...] = jnp.dot(oh, x_ref[...], preferred_element_type=jnp.float32)

  return pl.pallas_call(
      body,
      out_shape=jax.ShapeDtypeStruct((_G, _D), jnp.float32),
  )(xp, b4p)


def kernel(x, edge_index, batch, W_conv, b_conv, W_score, b_score):
  srcp = jnp.full((_EPAD,), _N, jnp.int32).at[:_E].set(edge_index[0])
  dstp = jnp.full((_EPAD,), _N, jnp.int32).at[:_E].set(edge_index[1])
  srcs, dsts = _shift_call(srcp.reshape(_ROWS, _C), dstp.reshape(_ROWS, _C))

  xp = jnp.zeros((_NPAD0, _D), jnp.float32).at[:_N].set(x)

  npad, n = _NPAD0, _N
  for t in range(_L):
    src_l = srcs[t]
    dst_l = dsts[t]
    chunk, _, _ = _acc_sched(npad)
    zer2 = jnp.zeros((chunk, _D), jnp.float32)
    npadc = 128 * (-(-npad // 128))
    zer1 = jnp.zeros((npadc,), jnp.float32)
    if t == 0:
      aggp = _sc_agg_call(npad, xp, src_l.reshape(_EPAD),
                          dst_l.reshape(_EPAD), zer2,
                          lean=True, stage=False, stream_dst=True,
                          single=True)
    elif t == 1:
      aggp = _sc_agg_call(npad, xp, src_l.reshape(_EPAD), dst_l, zer2,
                          lean=True, stage=True)
    else:
      aggp = _sc_agg_call(npad, xp, src_l, dst_l, zer2,
                          lean=False, stage=True)
    wsc = (jnp.zeros((_D, _D), jnp.float32)
           .at[:, 0].set(W_score[t, :_D])
           .at[:, 1].set(W_score[t, _D:]))
    bsr = jnp.zeros((1, _D), jnp.float32).at[0, 0].set(b_score[t])
    h, s12 = _dense1_call(xp, aggp, W_conv[t], b_conv[t][None, :], wsc, bsr, n)
    nwf = _sc_score(npad, s12[:, 0], s12[:, 1], src_l, dst_l, zer1)
    hr = h.reshape(npad // 2, 2 * _D)
    nwp = nwf.reshape(_NC, npadc)[:, :npad].reshape(_NC, npad // 2, 2)
    xp = _combine_call(hr, nwp)
    npad //= 2
    n //= 2

  b4 = batch[::2 ** _L]
  b4p = jnp.zeros((1, npad), jnp.int32).at[0, :b4.shape[0]].set(b4)
  return _pool_call(xp, b4p)


# speedup vs baseline: 1.0444x; 1.0444x over previous
import functools

import jax
import jax.numpy as jnp
from jax import lax
from jax.experimental import pallas as pl
from jax.experimental.pallas import tpu as pltpu
from jax.experimental.pallas import tpu_sc as plsc

_N, _E, _D, _L, _G = 10000, 320000, 128, 4, 16
_NC, _NS = 2, 16
_NW = _NC * _NS
_C = 128
_ROWS = 2560
_EPAD = _ROWS * _C
_RPT = _ROWS // _NW
_NB = 4
_NGRP = _RPT // _NB
_NPAD0 = 10112
_BN = 632
_BNE = 640


def _sc_mesh():
  return plsc.VectorSubcoreMesh(core_axis_name="c", subcore_axis_name="s",
                                num_cores=_NC, num_subcores=_NS)


_SC_PARAMS = pltpu.CompilerParams(needs_layout_passes=False)


def _shift_body(s_ref, d_ref, os_ref, od_ref):
  for t in range(_L):
    os_ref[t] = lax.shift_right_logical(s_ref[...], t)
    od_ref[t] = lax.shift_right_logical(d_ref[...], t)


def _shift_call(srcp, dstp):
  grid = _ROWS // _BNE
  return pl.pallas_call(
      _shift_body,
      grid=(grid,),
      in_specs=[pl.BlockSpec((_BNE, _C), lambda i: (i, 0)),
                pl.BlockSpec((_BNE, _C), lambda i: (i, 0))],
      out_specs=[pl.BlockSpec((_L, _BNE, _C), lambda i: (0, i, 0)),
                 pl.BlockSpec((_L, _BNE, _C), lambda i: (0, i, 0))],
      out_shape=[jax.ShapeDtypeStruct((_L, _ROWS, _C), jnp.int32),
                 jax.ShapeDtypeStruct((_L, _ROWS, _C), jnp.int32)],
  )(srcp, dstp)


def _acc_sched(npad):
  chunk = 8 * (-(-npad // (8 * _NS)))
  full = npad // chunk
  rem = npad - full * chunk
  return chunk, full, rem


def _acc_zero(zer_hbm, acc, sid, chunk, full, rem):
  @pl.when(sid < full)
  def _zero():
    pltpu.sync_copy(zer_hbm, acc.at[pl.ds(sid * chunk, chunk)])

  if rem:
    @pl.when(sid == full)
    def _zero_rem():
      pltpu.sync_copy(zer_hbm.at[pl.ds(0, rem)],
                      acc.at[pl.ds(full * chunk, rem)])


def _acc_out(acc, out_hbm, cid, sid, chunk, full, rem):
  @pl.when(sid < full)
  def _out():
    pltpu.sync_copy(acc.at[pl.ds(sid * chunk, chunk)],
                    out_hbm.at[cid, pl.ds(sid * chunk, chunk)])

  if rem:
    @pl.when(sid == full)
    def _out_rem():
      pltpu.sync_copy(acc.at[pl.ds(full * chunk, rem)],
                      out_hbm.at[cid, pl.ds(full * chunk, rem)])


def _stage_x(x_hbm, xsh, sid, chunk, full, rem):
  @pl.when(sid < full)
  def _st():
    pltpu.sync_copy(x_hbm.at[pl.ds(sid * chunk, chunk)],
                    xsh.at[pl.ds(sid * chunk, chunk)])

  if rem:
    @pl.when(sid == full)
    def _st_rem():
      pltpu.sync_copy(x_hbm.at[pl.ds(full * chunk, rem)],
                      xsh.at[pl.ds(full * chunk, rem)])


def _sc_agg_call(npad, xp, src_arr, dst_r, zer2, *, lean, stage,
                 split=None, stream_dst=False, single=False, delay_ns=0):
  chunk, full, rem = _acc_sched(npad)
  if lean:
    nb = 3 if stream_dst else 2
  else:
    nb = _NB
  rpt = _ROWS // _NS if single else _RPT
  rpt_max = split[0] if split else rpt
  nc_out = 1 if single else _NC

  scratch = []
  if lean:
    if not stream_dst:
      scratch += [pltpu.VMEM((rpt_max, _C), jnp.int32)]
    scratch += [pltpu.VMEM((_C,), jnp.int32)] * nb
    if stream_dst:
      scratch += [pltpu.VMEM((_C,), jnp.int32)] * nb
  else:
    scratch += [pltpu.VMEM((rpt, _C), jnp.int32),
                pltpu.VMEM((rpt, _C), jnp.int32)]
  scratch += [pltpu.VMEM((_C, _D), jnp.float32)] * nb
  scratch += [pltpu.VMEM_SHARED((npad, _D), jnp.float32)]
  if stage:
    scratch += [pltpu.VMEM_SHARED((npad, _D), jnp.float32)]
  scratch += [pltpu.SemaphoreType.DMA] * ((3 if lean else 2) * nb)

  @functools.partial(
      pl.kernel,
      out_type=jax.ShapeDtypeStruct((nc_out, npad, _D), jnp.float32),
      mesh=_sc_mesh(),
      compiler_params=_SC_PARAMS,
      scratch_types=scratch)
  def k(x_hbm, src_hbm, dst_hbm, zer_hbm, out_hbm, *rest):
    p = 0
    didx = di = sidx = None
    if lean:
      if not stream_dst:
        didx = rest[p]; p += 1
      si = rest[p:p + nb]; p += nb
      if stream_dst:
        di = rest[p:p + nb]; p += nb
    else:
      sidx, didx = rest[p], rest[p + 1]; p += 2
    rows = rest[p:p + nb]; p += nb
    acc = rest[p]; p += 1
    xsh = rest[p] if stage else None
    if stage:
      p += 1
    sems = rest[p:]
    if lean:
      isem, gsem, ssem = sems[:nb], sems[nb:2 * nb], sems[2 * nb:]
    else:
      gsem, ssem = sems[:nb], sems[nb:]

    cid = lax.axis_index("c")
    sid = lax.axis_index("s")
    wid = sid * _NC + cid

    if split:
      rpt_f, rpt_s, fcid = split
      is_fast = cid == fcid
      rpt_my = jnp.where(is_fast, rpt_f, rpt_s)
      base_e = jnp.where(is_fast, sid * rpt_f, _NS * rpt_f + sid * rpt_s)
      ngrp_my = (rpt_my + nb - 1) // nb
      ragged = True
    elif single:
      rpt_my = rpt
      base_e = sid * rpt
      ngrp_my = (rpt + nb - 1) // nb
      ragged = rpt % nb != 0
    else:
      rpt_my = rpt
      base_e = wid * rpt
      ngrp_my = (rpt + nb - 1) // nb
      ragged = rpt % nb != 0

    def dref(g, b):
      return di[b] if stream_dst else didx.at[g]

    def scat_wait(b):
      d = di[b] if stream_dst else didx.at[0]
      pltpu.make_async_copy(rows[b], acc.at[d], ssem[b]).wait()

    def work():
      with jax.named_scope("agg_prep"):
        if didx is not None:
          pltpu.sync_copy(dst_hbm.at[pl.ds(base_e, rpt)], didx)
        if not lean:
          pltpu.sync_copy(src_hbm.at[pl.ds(base_e, rpt)], sidx)
        _acc_zero(zer_hbm, acc, sid, chunk, full, rem)
        if stage:
          _stage_x(x_hbm, xsh, sid, chunk, full, rem)
        plsc.subcore_barrier()
      gsrc = xsh if stage else x_hbm

      if lean:
        def idx_issue(g, b):
          pltpu.async_copy(src_hbm.at[pl.ds((base_e + g) * _C, _C)], si[b],
                           isem[b])
          if stream_dst:
            pltpu.async_copy(dst_hbm.at[pl.ds((base_e + g) * _C, _C)],
                             di[b], isem[b])

        def idx_wait(b):
          pltpu.make_async_copy(src_hbm.at[pl.ds(0, _C)], si[b],
                                isem[b]).wait()
          if stream_dst:
            pltpu.make_async_copy(src_hbm.at[pl.ds(0, _C)], di[b],
                                  isem[b]).wait()

        for b in range(nb):
          idx_issue(b, b)
        for b in range(nb):
          idx_wait(b)
          pltpu.async_copy(gsrc.at[si[b]], rows[b], gsem[b]).wait()
          idx_issue(b + nb, b)
          pltpu.async_copy(rows[b], acc.at[dref(b, b)], ssem[b], add=True)

        def group(k_i, carry):
          for b in range(nb):
            g = nb * k_i + b

            def slot_body():
              scat_wait(b)
              idx_wait(b)
              pltpu.async_copy(gsrc.at[si[b]], rows[b], gsem[b]).wait()
              g_next = g + nb

              @pl.when(g_next < rpt_my)
              def _prefetch_idx():
                idx_issue(g_next, b)

              pltpu.async_copy(rows[b], acc.at[dref(g, b)], ssem[b],
                               add=True)

            if ragged:
              pl.when(g < rpt_my)(slot_body)
            else:
              slot_body()
          return carry

        lax.fori_loop(1, ngrp_my, group, 0)
        for b in range(nb):
          scat_wait(b)
      else:
        for b in range(nb):
          pltpu.async_copy(gsrc.at[sidx.at[b]], rows[b], gsem[b])

        def group(k_i, carry):
          scat = []
          for b in range(nb):
            g = nb * k_i + b
            pltpu.make_async_copy(gsrc.at[sidx.at[g]], rows[b],
                                  gsem[b]).wait()
            scat.append(
                pltpu.async_copy(rows[b], acc.at[didx.at[g]], ssem[b],
                                 add=True))
          for b in range(nb):
            g_next = nb * k_i + b + nb
            scat[b].wait()

            @pl.when(g_next < rpt)
            def _prefetch():
              pltpu.async_copy(gsrc.at[sidx.at[g_next]], rows[b], gsem[b])

          return carry

        lax.fori_loop(0, ngrp_my, group, 0)

      with jax.named_scope("agg_out"):
        if split and delay_ns:
          @pl.when(cid != split[2])
          def _hold():
            pl.delay(delay_ns)

        plsc.subcore_barrier()
        _acc_out(acc, out_hbm, 0 if single else cid, sid, chunk, full, rem)

    if single:
      pl.when(cid == 0)(work)
    else:
      work()

  return k(xp, src_arr, dst_r, zer2)


def _dense1_call(xp, aggp, wt, bt, wsc, bsr, nreal):
  npad = xp.shape[0]
  grid = npad // _BN

  def body(x_ref, a_ref, w_ref, b_ref, ws_ref, bs_ref, h_ref, s_ref):
    i = pl.program_id(0)
    xa = x_ref[...]
    for q in range(a_ref.shape[0]):
      xa = xa + a_ref[q]
    h = jnp.dot(xa, w_ref[...], preferred_element_type=jnp.float32)
    h = jnp.maximum(h + b_ref[...], 0.0)
    rows = i * _BN + lax.broadcasted_iota(jnp.int32, (_BN, 1), 0)
    h = jnp.where(rows < nreal, h, 0.0)
    h_ref[...] = h
    s_ref[...] = jnp.dot(h, ws_ref[...],
                         preferred_element_type=jnp.float32) + bs_ref[...]

  return pl.pallas_call(
      body,
      grid=(grid,),
      in_specs=[pl.BlockSpec((_BN, _D), lambda i: (i, 0)),
                pl.BlockSpec((aggp.shape[0], _BN, _D), lambda i: (0, i, 0)),
                pl.BlockSpec((_D, _D), lambda i: (0, 0)),
                pl.BlockSpec((1, _D), lambda i: (0, 0)),
                pl.BlockSpec((_D, _D), lambda i: (0, 0)),
                pl.BlockSpec((1, _D), lambda i: (0, 0))],
      out_specs=[pl.BlockSpec((_BN, _D), lambda i: (i, 0)),
                 pl.BlockSpec((_BN, _D), lambda i: (i, 0))],
      out_shape=[jax.ShapeDtypeStruct((npad, _D), jnp.float32),
                 jax.ShapeDtypeStruct((npad, _D), jnp.float32)],
  )(xp, aggp, wt, bt, wsc, bsr)


def _sc_score(npad, s1, s2, src_l, dst_l, zer1):
  npadc = 128 * (-(-npad // 128))

  @functools.partial(
      pl.kernel,
      out_type=jax.ShapeDtypeStruct((_NC * npadc,), jnp.float32),
      mesh=_sc_mesh(),
      compiler_params=_SC_PARAMS,
      scratch_types=[
          pltpu.VMEM((npad,), jnp.float32),
          pltpu.VMEM((npad,), jnp.float32),
          pltpu.VMEM((_RPT, _C), jnp.int32),
          pltpu.VMEM((_RPT, _C), jnp.int32),
          pltpu.VMEM((_RPT, _C), jnp.float32),
          pltpu.VMEM_SHARED((npadc,), jnp.float32),
          pltpu.SemaphoreType.DMA,
      ])
  def k(s1_hbm, s2_hbm, src_hbm, dst_hbm, zer_hbm, out_hbm,
        s1v, s2v, sidx, didx, sig, acc, ssem):
    cid = lax.axis_index("c")
    sid = lax.axis_index("s")
    wid = sid * _NC + cid
    pltpu.sync_copy(s1_hbm, s1v)
    pltpu.sync_copy(s2_hbm, s2v)
    pltpu.sync_copy(src_hbm.at[pl.ds(wid * _RPT, _RPT)], sidx)
    pltpu.sync_copy(dst_hbm.at[pl.ds(wid * _RPT, _RPT)], didx)

    @pl.when(sid == 0)
    def _zero():
      pltpu.sync_copy(zer_hbm, acc)

    plsc.subcore_barrier()

    def body(i, carry):
      for j in range(_C // 16):
        s_idx = sidx[i, pl.ds(16 * j, 16)]
        d_idx = didx[i, pl.ds(16 * j, 16)]
        v1 = plsc.load_gather(s1v, [s_idx])
        v2 = plsc.load_gather(s2v, [d_idx])
        z = v1 + v2
        sig[i, pl.ds(16 * j, 16)] = 1.0 / (1.0 + jnp.exp(-z))
      pltpu.async_copy(sig.at[i], acc.at[didx.at[i]], ssem, add=True)
      return carry

    lax.fori_loop(0, _RPT, body, 0)

    def drain(i, carry):
      pltpu.make_async_copy(sig.at[0], acc.at[didx.at[0]], ssem).wait()
      return carry

    lax.fori_loop(0, _RPT, drain, 0)
    plsc.subcore_barrier()

    @pl.when(sid == 0)
    def _out():
      pltpu.sync_copy(acc, out_hbm.at[pl.ds(cid * npadc, npadc)])

  return k(s1, s2, src_l, dst_l, zer1)


def _combine_call(hr, nwp):
  n2 = hr.shape[0]
  grid = n2 // _BN

  def body(h_ref, nw_ref, o_ref):
    nw = nw_ref[0] + nw_ref[1]
    w0 = 1.0 + nw[:, 0:1]
    w1 = 1.0 + nw[:, 1:2]
    o_ref[...] = h_ref[:, :_D] * w0 + h_ref[:, _D:] * w1

  return pl.pallas_call(
      body,
      grid=(grid,),
      in_specs=[pl.BlockSpec((_BN, 2 * _D), lambda i: (i, 0)),
                pl.BlockSpec((_NC, _BN, 2), lambda i: (0, i, 0))],
      out_specs=pl.BlockSpec((_BN, _D), lambda i: (i, 0)),
      out_shape=jax.ShapeDtypeStruct((n2, _D), jnp.float32),
  )(hr, nwp)


def _pool_call(xp, b4p):
  n4 = xp.shape[0]

  def body(x_ref, b_ref, o_ref):
    iota = lax.broadcasted_iota(jnp.int32, (_G, n4), 0)
    oh = (b_ref[...] == iota).astype(jnp.float32)
    o_ref[...] = jnp.dot(oh, x_ref[...], preferred_element_type=jnp.float32)

  return pl.pallas_call(
      body,
      out_shape=jax.ShapeDtypeStruct((_G, _D), jnp.float32),
  )(xp, b4p)


def kernel(x, edge_index, batch, W_conv, b_conv, W_score, b_score):
  srcp = jnp.full((_EPAD,), _N, jnp.int32).at[:_E].set(edge_index[0])
  dstp = jnp.full((_EPAD,), _N, jnp.int32).at[:_E].set(edge_index[1])
  srcs, dsts = _shift_call(srcp.reshape(_ROWS, _C), dstp.reshape(_ROWS, _C))

  xp = jnp.zeros((_NPAD0, _D), jnp.float32).at[:_N].set(x)

  npad, n = _NPAD0, _N
  for t in range(_L):
    src_l = srcs[t]
    dst_l = dsts[t]
    chunk, _, _ = _acc_sched(npad)
    zer2 = jnp.zeros((chunk, _D), jnp.float32)
    npadc = 128 * (-(-npad // 128))
    zer1 = jnp.zeros((npadc,), jnp.float32)
    if t == 0:
      aggp = _sc_agg_call(npad, xp, src_l.reshape(_EPAD),
                          dst_l.reshape(_EPAD), zer2,
                          lean=True, stage=False, stream_dst=True,
                          split=(112, 48, 0), delay_ns=120_000)
    elif t == 1:
      aggp = _sc_agg_call(npad, xp, src_l.reshape(_EPAD), dst_l, zer2,
                          lean=True, stage=True)
    else:
      aggp = _sc_agg_call(npad, xp, src_l, dst_l, zer2,
                          lean=False, stage=True)
    wsc = (jnp.zeros((_D, _D), jnp.float32)
           .at[:, 0].set(W_score[t, :_D])
           .at[:, 1].set(W_score[t, _D:]))
    bsr = jnp.zeros((1, _D), jnp.float32).at[0, 0].set(b_score[t])
    h, s12 = _dense1_call(xp, aggp, W_conv[t], b_conv[t][None, :], wsc, bsr, n)
    nwf = _sc_score(npad, s12[:, 0], s12[:, 1], src_l, dst_l, zer1)
    hr = h.reshape(npad // 2, 2 * _D)
    nwp = nwf.reshape(_NC, npadc)[:, :npad].reshape(_NC, npad // 2, 2)
    xp = _combine_call(hr, nwp)
    npad //= 2
    n //= 2

  b4 = batch[::2 ** _L]
  b4p = jnp.zeros((1, npad), jnp.int32).at[0, :b4.shape[0]].set(b4)
  return _pool_call(xp, b4p)


# speedup vs baseline: 1.1895x; 1.1389x over previous
import functools

import jax
import jax.numpy as jnp
from jax import lax
from jax.experimental import pallas as pl
from jax.experimental.pallas import tpu as pltpu
from jax.experimental.pallas import tpu_sc as plsc

_N, _E, _D, _L, _G = 10000, 320000, 128, 4, 16
_NC, _NS = 2, 16
_NW = _NC * _NS
_C = 128
_ROWS = 2560
_EPAD = _ROWS * _C
_RPT = _ROWS // _NW
_NB = 4
_NGRP = _RPT // _NB
_NPAD0 = 10112
_BN = 632
_BNE = 640


def _sc_mesh():
  return plsc.VectorSubcoreMesh(core_axis_name="c", subcore_axis_name="s",
                                num_cores=_NC, num_subcores=_NS)


_SC_PARAMS = pltpu.CompilerParams(needs_layout_passes=False)


def _shift_body(s_ref, d_ref, os_ref, od_ref):
  for t in range(_L):
    os_ref[t] = lax.shift_right_logical(s_ref[...], t)
    od_ref[t] = lax.shift_right_logical(d_ref[...], t)


def _shift_call(srcp, dstp):
  grid = _ROWS // _BNE
  return pl.pallas_call(
      _shift_body,
      grid=(grid,),
      in_specs=[pl.BlockSpec((_BNE, _C), lambda i: (i, 0)),
                pl.BlockSpec((_BNE, _C), lambda i: (i, 0))],
      out_specs=[pl.BlockSpec((_L, _BNE, _C), lambda i: (0, i, 0)),
                 pl.BlockSpec((_L, _BNE, _C), lambda i: (0, i, 0))],
      out_shape=[jax.ShapeDtypeStruct((_L, _ROWS, _C), jnp.int32),
                 jax.ShapeDtypeStruct((_L, _ROWS, _C), jnp.int32)],
  )(srcp, dstp)


def _acc_sched(npad):
  chunk = 8 * (-(-npad // (8 * _NS)))
  full = npad // chunk
  rem = npad - full * chunk
  return chunk, full, rem


def _acc_zero(zer_hbm, acc, sid, chunk, full, rem):
  @pl.when(sid < full)
  def _zero():
    pltpu.sync_copy(zer_hbm, acc.at[pl.ds(sid * chunk, chunk)])

  if rem:
    @pl.when(sid == full)
    def _zero_rem():
      pltpu.sync_copy(zer_hbm.at[pl.ds(0, rem)],
                      acc.at[pl.ds(full * chunk, rem)])


def _acc_out(acc, out_hbm, cid, sid, chunk, full, rem):
  @pl.when(sid < full)
  def _out():
    pltpu.sync_copy(acc.at[pl.ds(sid * chunk, chunk)],
                    out_hbm.at[cid, pl.ds(sid * chunk, chunk)])

  if rem:
    @pl.when(sid == full)
    def _out_rem():
      pltpu.sync_copy(acc.at[pl.ds(full * chunk, rem)],
                      out_hbm.at[cid, pl.ds(full * chunk, rem)])


def _stage_x(x_hbm, xsh, sid, chunk, full, rem):
  @pl.when(sid < full)
  def _st():
    pltpu.sync_copy(x_hbm.at[pl.ds(sid * chunk, chunk)],
                    xsh.at[pl.ds(sid * chunk, chunk)])

  if rem:
    @pl.when(sid == full)
    def _st_rem():
      pltpu.sync_copy(x_hbm.at[pl.ds(full * chunk, rem)],
                      xsh.at[pl.ds(full * chunk, rem)])


def _sc_agg_call(npad, xp, src_arr, dst_r, zer2, *, lean, stage,
                 split=None, stream_dst=False, single=False, delay_ns=0):
  chunk, full, rem = _acc_sched(npad)
  if lean:
    nb = 3 if stream_dst else 2
  else:
    nb = _NB
  rpt = _ROWS // _NS if single else _RPT
  rpt_max = split[0] if split else rpt
  nc_out = 1 if single else _NC

  scratch = []
  if lean:
    if not stream_dst:
      scratch += [pltpu.VMEM((rpt_max, _C), jnp.int32)]
    scratch += [pltpu.VMEM((_C,), jnp.int32)] * nb
    if stream_dst:
      scratch += [pltpu.VMEM((_C,), jnp.int32)] * nb
  else:
    scratch += [pltpu.VMEM((rpt, _C), jnp.int32),
                pltpu.VMEM((rpt, _C), jnp.int32)]
  scratch += [pltpu.VMEM((_C, _D), jnp.float32)] * nb
  scratch += [pltpu.VMEM_SHARED((npad, _D), jnp.float32)]
  if stage:
    scratch += [pltpu.VMEM_SHARED((npad, _D), jnp.float32)]
  scratch += [pltpu.SemaphoreType.DMA] * ((3 if lean else 2) * nb)

  @functools.partial(
      pl.kernel,
      out_type=jax.ShapeDtypeStruct((nc_out, npad, _D), jnp.float32),
      mesh=_sc_mesh(),
      compiler_params=_SC_PARAMS,
      scratch_types=scratch)
  def k(x_hbm, src_hbm, dst_hbm, zer_hbm, out_hbm, *rest):
    p = 0
    didx = di = sidx = None
    if lean:
      if not stream_dst:
        didx = rest[p]; p += 1
      si = rest[p:p + nb]; p += nb
      if stream_dst:
        di = rest[p:p + nb]; p += nb
    else:
      sidx, didx = rest[p], rest[p + 1]; p += 2
    rows = rest[p:p + nb]; p += nb
    acc = rest[p]; p += 1
    xsh = rest[p] if stage else None
    if stage:
      p += 1
    sems = rest[p:]
    if lean:
      isem, gsem, ssem = sems[:nb], sems[nb:2 * nb], sems[2 * nb:]
    else:
      gsem, ssem = sems[:nb], sems[nb:]

    cid = lax.axis_index("c")
    sid = lax.axis_index("s")
    wid = sid * _NC + cid

    if split:
      rpt_f, rpt_s, fcid = split
      is_fast = cid == fcid
      rpt_my = jnp.where(is_fast, rpt_f, rpt_s)
      base_e = jnp.where(is_fast, sid * rpt_f, _NS * rpt_f + sid * rpt_s)
      ngrp_my = (rpt_my + nb - 1) // nb
      ragged = True
    elif single:
      rpt_my = rpt
      base_e = sid * rpt
      ngrp_my = (rpt + nb - 1) // nb
      ragged = rpt % nb != 0
    else:
      rpt_my = rpt
      base_e = wid * rpt
      ngrp_my = (rpt + nb - 1) // nb
      ragged = rpt % nb != 0

    def dref(g, b):
      return di[b] if stream_dst else didx.at[g]

    def scat_wait(b):
      d = di[b] if stream_dst else didx.at[0]
      pltpu.make_async_copy(rows[b], acc.at[d], ssem[b]).wait()

    def work():
      with jax.named_scope("agg_prep"):
        if didx is not None:
          if split:
            @pl.when(is_fast)
            def _pre_f():
              pltpu.sync_copy(dst_hbm.at[pl.ds(base_e, split[0])],
                              didx.at[pl.ds(0, split[0])])

            @pl.when(jnp.logical_not(is_fast))
            def _pre_s():
              pltpu.sync_copy(dst_hbm.at[pl.ds(base_e, split[1])],
                              didx.at[pl.ds(0, split[1])])
          else:
            pltpu.sync_copy(dst_hbm.at[pl.ds(base_e, rpt)], didx)
        if not lean:
          pltpu.sync_copy(src_hbm.at[pl.ds(base_e, rpt)], sidx)
        _acc_zero(zer_hbm, acc, sid, chunk, full, rem)
        if stage:
          _stage_x(x_hbm, xsh, sid, chunk, full, rem)
        plsc.subcore_barrier()
      gsrc = xsh if stage else x_hbm

      if lean:
        def idx_issue(g, b):
          pltpu.async_copy(src_hbm.at[pl.ds((base_e + g) * _C, _C)], si[b],
                           isem[b])
          if stream_dst:
            pltpu.async_copy(dst_hbm.at[pl.ds((base_e + g) * _C, _C)],
                             di[b], isem[b])

        def idx_wait(b):
          pltpu.make_async_copy(src_hbm.at[pl.ds(0, _C)], si[b],
                                isem[b]).wait()
          if stream_dst:
            pltpu.make_async_copy(src_hbm.at[pl.ds(0, _C)], di[b],
                                  isem[b]).wait()

        for b in range(nb):
          idx_issue(b, b)
        for b in range(nb):
          idx_wait(b)
          pltpu.async_copy(gsrc.at[si[b]], rows[b], gsem[b]).wait()
          idx_issue(b + nb, b)
          pltpu.async_copy(rows[b], acc.at[dref(b, b)], ssem[b], add=True)

        def group(k_i, carry):
          for b in range(nb):
            g = nb * k_i + b

            def slot_body():
              scat_wait(b)
              idx_wait(b)
              pltpu.async_copy(gsrc.at[si[b]], rows[b], gsem[b]).wait()
              g_next = g + nb

              @pl.when(g_next < rpt_my)
              def _prefetch_idx():
                idx_issue(g_next, b)

              pltpu.async_copy(rows[b], acc.at[dref(g, b)], ssem[b],
                               add=True)

            if ragged:
              pl.when(g < rpt_my)(slot_body)
            else:
              slot_body()
          return carry

        lax.fori_loop(1, ngrp_my, group, 0)
        for b in range(nb):
          scat_wait(b)
      else:
        for b in range(nb):
          pltpu.async_copy(gsrc.at[sidx.at[b]], rows[b], gsem[b])

        def group(k_i, carry):
          scat = []
          for b in range(nb):
            g = nb * k_i + b
            pltpu.make_async_copy(gsrc.at[sidx.at[g]], rows[b],
                                  gsem[b]).wait()
            scat.append(
                pltpu.async_copy(rows[b], acc.at[didx.at[g]], ssem[b],
                                 add=True))
          for b in range(nb):
            g_next = nb * k_i + b + nb
            scat[b].wait()

            @pl.when(g_next < rpt)
            def _prefetch():
              pltpu.async_copy(gsrc.at[sidx.at[g_next]], rows[b], gsem[b])

          return carry

        lax.fori_loop(0, ngrp_my, group, 0)

      with jax.named_scope("agg_out"):
        if split and delay_ns:
          @pl.when(cid != split[2])
          def _hold():
            pl.delay(delay_ns)

        plsc.subcore_barrier()
        _acc_out(acc, out_hbm, 0 if single else cid, sid, chunk, full, rem)

    if single:
      pl.when(cid == 0)(work)
    else:
      work()

  return k(xp, src_arr, dst_r, zer2)


def _dense1_call(xp, aggp, wt, bt, wsc, bsr, nreal):
  npad = xp.shape[0]
  grid = npad // _BN

  def body(x_ref, a_ref, w_ref, b_ref, ws_ref, bs_ref, h_ref, s_ref):
    i = pl.program_id(0)
    xa = x_ref[...]
    for q in range(a_ref.shape[0]):
      xa = xa + a_ref[q]
    h = jnp.dot(xa, w_ref[...], preferred_element_type=jnp.float32)
    h = jnp.maximum(h + b_ref[...], 0.0)
    rows = i * _BN + lax.broadcasted_iota(jnp.int32, (_BN, 1), 0)
    h = jnp.where(rows < nreal, h, 0.0)
    h_ref[...] = h
    s_ref[...] = jnp.dot(h, ws_ref[...],
                         preferred_element_type=jnp.float32) + bs_ref[...]

  return pl.pallas_call(
      body,
      grid=(grid,),
      in_specs=[pl.BlockSpec((_BN, _D), lambda i: (i, 0)),
                pl.BlockSpec((aggp.shape[0], _BN, _D), lambda i: (0, i, 0)),
                pl.BlockSpec((_D, _D), lambda i: (0, 0)),
                pl.BlockSpec((1, _D), lambda i: (0, 0)),
                pl.BlockSpec((_D, _D), lambda i: (0, 0)),
                pl.BlockSpec((1, _D), lambda i: (0, 0))],
      out_specs=[pl.BlockSpec((_BN, _D), lambda i: (i, 0)),
                 pl.BlockSpec((_BN, _D), lambda i: (i, 0))],
      out_shape=[jax.ShapeDtypeStruct((npad, _D), jnp.float32),
                 jax.ShapeDtypeStruct((npad, _D), jnp.float32)],
  )(xp, aggp, wt, bt, wsc, bsr)


def _sc_score(npad, s1, s2, src_l, dst_l, zer1):
  npadc = 128 * (-(-npad // 128))

  @functools.partial(
      pl.kernel,
      out_type=jax.ShapeDtypeStruct((_NC * npadc,), jnp.float32),
      mesh=_sc_mesh(),
      compiler_params=_SC_PARAMS,
      scratch_types=[
          pltpu.VMEM((npad,), jnp.float32),
          pltpu.VMEM((npad,), jnp.float32),
          pltpu.VMEM((_RPT, _C), jnp.int32),
          pltpu.VMEM((_RPT, _C), jnp.int32),
          pltpu.VMEM((_RPT, _C), jnp.float32),
          pltpu.VMEM_SHARED((npadc,), jnp.float32),
          pltpu.SemaphoreType.DMA,
      ])
  def k(s1_hbm, s2_hbm, src_hbm, dst_hbm, zer_hbm, out_hbm,
        s1v, s2v, sidx, didx, sig, acc, ssem):
    cid = lax.axis_index("c")
    sid = lax.axis_index("s")
    wid = sid * _NC + cid
    pltpu.sync_copy(s1_hbm, s1v)
    pltpu.sync_copy(s2_hbm, s2v)
    pltpu.sync_copy(src_hbm.at[pl.ds(wid * _RPT, _RPT)], sidx)
    pltpu.sync_copy(dst_hbm.at[pl.ds(wid * _RPT, _RPT)], didx)

    @pl.when(sid == 0)
    def _zero():
      pltpu.sync_copy(zer_hbm, acc)

    plsc.subcore_barrier()

    def body(i, carry):
      for j in range(_C // 16):
        s_idx = sidx[i, pl.ds(16 * j, 16)]
        d_idx = didx[i, pl.ds(16 * j, 16)]
        v1 = plsc.load_gather(s1v, [s_idx])
        v2 = plsc.load_gather(s2v, [d_idx])
        z = v1 + v2
        sig[i, pl.ds(16 * j, 16)] = 1.0 / (1.0 + jnp.exp(-z))
      pltpu.async_copy(sig.at[i], acc.at[didx.at[i]], ssem, add=True)
      return carry

    lax.fori_loop(0, _RPT, body, 0)

    def drain(i, carry):
      pltpu.make_async_copy(sig.at[0], acc.at[didx.at[0]], ssem).wait()
      return carry

    lax.fori_loop(0, _RPT, drain, 0)
    plsc.subcore_barrier()

    @pl.when(sid == 0)
    def _out():
      pltpu.sync_copy(acc, out_hbm.at[pl.ds(cid * npadc, npadc)])

  return k(s1, s2, src_l, dst_l, zer1)


def _combine_call(hr, nwp):
  n2 = hr.shape[0]
  grid = n2 // _BN

  def body(h_ref, nw_ref, o_ref):
    nw = nw_ref[0] + nw_ref[1]
    w0 = 1.0 + nw[:, 0:1]
    w1 = 1.0 + nw[:, 1:2]
    o_ref[...] = h_ref[:, :_D] * w0 + h_ref[:, _D:] * w1

  return pl.pallas_call(
      body,
      grid=(grid,),
      in_specs=[pl.BlockSpec((_BN, 2 * _D), lambda i: (i, 0)),
                pl.BlockSpec((_NC, _BN, 2), lambda i: (0, i, 0))],
      out_specs=pl.BlockSpec((_BN, _D), lambda i: (i, 0)),
      out_shape=jax.ShapeDtypeStruct((n2, _D), jnp.float32),
  )(hr, nwp)


def _pool_call(xp, b4p):
  n4 = xp.shape[0]

  def body(x_ref, b_ref, o_ref):
    iota = lax.broadcasted_iota(jnp.int32, (_G, n4), 0)
    oh = (b_ref[...] == iota).astype(jnp.float32)
    o_ref[...] = jnp.dot(oh, x_ref[...], preferred_element_type=jnp.float32)

  return pl.pallas_call(
      body,
      out_shape=jax.ShapeDtypeStruct((_G, _D), jnp.float32),
  )(xp, b4p)


def kernel(x, edge_index, batch, W_conv, b_conv, W_score, b_score):
  srcp = jnp.full((_EPAD,), _N, jnp.int32).at[:_E].set(edge_index[0])
  dstp = jnp.full((_EPAD,), _N, jnp.int32).at[:_E].set(edge_index[1])
  srcs, dsts = _shift_call(srcp.reshape(_ROWS, _C), dstp.reshape(_ROWS, _C))

  xp = jnp.zeros((_NPAD0, _D), jnp.float32).at[:_N].set(x)

  npad, n = _NPAD0, _N
  for t in range(_L):
    src_l = srcs[t]
    dst_l = dsts[t]
    chunk, _, _ = _acc_sched(npad)
    zer2 = jnp.zeros((chunk, _D), jnp.float32)
    npadc = 128 * (-(-npad // 128))
    zer1 = jnp.zeros((npadc,), jnp.float32)
    if t == 0:
      aggp = _sc_agg_call(npad, xp, src_l.reshape(_EPAD), dst_l, zer2,
                          lean=True, stage=False, split=(120, 40, 0))
    elif t == 1:
      aggp = _sc_agg_call(npad, xp, src_l.reshape(_EPAD), dst_l, zer2,
                          lean=True, stage=True)
    else:
      aggp = _sc_agg_call(npad, xp, src_l, dst_l, zer2,
                          lean=False, stage=True)
    wsc = (jnp.zeros((_D, _D), jnp.float32)
           .at[:, 0].set(W_score[t, :_D])
           .at[:, 1].set(W_score[t, _D:]))
    bsr = jnp.zeros((1, _D), jnp.float32).at[0, 0].set(b_score[t])
    h, s12 = _dense1_call(xp, aggp, W_conv[t], b_conv[t][None, :], wsc, bsr, n)
    nwf = _sc_score(npad, s12[:, 0], s12[:, 1], src_l, dst_l, zer1)
    hr = h.reshape(npad // 2, 2 * _D)
    nwp = nwf.reshape(_NC, npadc)[:, :npad].reshape(_NC, npad // 2, 2)
    xp = _combine_call(hr, nwp)
    npad //= 2
    n //= 2

  b4 = batch[::2 ** _L]
  b4p = jnp.zeros((1, npad), jnp.int32).at[0, :b4.shape[0]].set(b4)
  return _pool_call(xp, b4p)
